# baseline (speedup 1.0000x reference)
import jax
import jax.numpy as jnp
from jax.experimental import pallas as pl
from jax.experimental.pallas import tpu as pltpu

_N = 10000
_BM = 400


def _pass1_kernel(x_ref, w1_ref, b1_ref, w2_ref, adj_ref, s2_ref, s1_scr):
    @pl.when(pl.program_id(0) == 0)
    def _():
        s1_scr[...] = jnp.dot(
            x_ref[...], w1_ref[...], preferred_element_type=jnp.float32
        )

    h = jnp.dot(adj_ref[...], s1_scr[...], preferred_element_type=jnp.float32)
    h = jnp.maximum(h + b1_ref[...], 0.0)
    s2_ref[...] = jnp.dot(h, w2_ref[...], preferred_element_type=jnp.float32)


def _pass2_kernel(s2_ref, b2_ref, adj_ref, out_ref):
    t = (
        jnp.dot(adj_ref[...], s2_ref[...], preferred_element_type=jnp.float32)
        + b2_ref[...]
    )
    m = jnp.max(t, axis=1, keepdims=True)
    lse = jnp.log(jnp.sum(jnp.exp(t - m), axis=1, keepdims=True)) + m
    out_ref[...] = t - lse


def kernel(x, adj, W1, b1, W2, b2):
    n, nfeat = x.shape
    nhid = W1.shape[1]
    nclass = W2.shape[1]
    grid = (n // _BM,)
    const = lambda m: (0, 0)
    rows = lambda m: (m, 0)

    s2 = pl.pallas_call(
        _pass1_kernel,
        grid=grid,
        in_specs=[
            pl.BlockSpec((n, nfeat), const),
            pl.BlockSpec((nfeat, nhid), const),
            pl.BlockSpec((1, nhid), const),
            pl.BlockSpec((nhid, nclass), const),
            pl.BlockSpec((_BM, n), rows),
        ],
        out_specs=pl.BlockSpec((_BM, nclass), rows),
        out_shape=jax.ShapeDtypeStruct((n, nclass), jnp.float32),
        scratch_shapes=[pltpu.VMEM((n, nhid), jnp.float32)],
        compiler_params=pltpu.CompilerParams(
            dimension_semantics=("arbitrary",),
        ),
    )(x, W1, b1.reshape(1, -1), W2, adj)

    out = pl.pallas_call(
        _pass2_kernel,
        grid=grid,
        in_specs=[
            pl.BlockSpec((n, nclass), const),
            pl.BlockSpec((1, nclass), const),
            pl.BlockSpec((_BM, n), rows),
        ],
        out_specs=pl.BlockSpec((_BM, nclass), rows),
        out_shape=jax.ShapeDtypeStruct((n, nclass), jnp.float32),
        compiler_params=pltpu.CompilerParams(
            dimension_semantics=("arbitrary",),
        ),
    )(s2, b2.reshape(1, -1), adj)

    return out


# speedup vs baseline: 1.0257x; 1.0257x over previous
import jax
import jax.numpy as jnp
from jax.experimental import pallas as pl
from jax.experimental.pallas import tpu as pltpu

_BM = 400


def _gcn_kernel(
    x_ref, w1_ref, b1_ref, w2_ref, b2_ref, adj_ref, out_ref, s1_scr, s2_scr
):
    p = pl.program_id(0)
    m = pl.program_id(1)

    @pl.when((p == 0) & (m == 0))
    def _():
        s1_scr[...] = jnp.dot(
            x_ref[...], w1_ref[...], preferred_element_type=jnp.float32
        )

    @pl.when(p == 0)
    def _():
        h = jnp.dot(
            adj_ref[...], s1_scr[...], preferred_element_type=jnp.float32
        )
        h = jnp.maximum(h + b1_ref[...], 0.0)
        s2_scr[pl.ds(m * _BM, _BM), :] = jnp.dot(
            h, w2_ref[...], preferred_element_type=jnp.float32
        )

    @pl.when(p == 1)
    def _():
        t = (
            jnp.dot(
                adj_ref[...], s2_scr[...], preferred_element_type=jnp.float32
            )
            + b2_ref[...]
        )
        mx = jnp.max(t, axis=1, keepdims=True)
        lse = jnp.log(jnp.sum(jnp.exp(t - mx), axis=1, keepdims=True)) + mx
        out_ref[...] = t - lse


def kernel(x, adj, W1, b1, W2, b2):
    n, nfeat = x.shape
    nhid = W1.shape[1]
    nclass = W2.shape[1]
    const = lambda p, m: (0, 0)
    rows = lambda p, m: (m, 0)

    return pl.pallas_call(
        _gcn_kernel,
        grid=(2, n // _BM),
        in_specs=[
            pl.BlockSpec((n, nfeat), const),
            pl.BlockSpec((nfeat, nhid), const),
            pl.BlockSpec((1, nhid), const),
            pl.BlockSpec((nhid, nclass), const),
            pl.BlockSpec((1, nclass), const),
            pl.BlockSpec((_BM, n), rows),
        ],
        out_specs=pl.BlockSpec((_BM, nclass), rows),
        out_shape=jax.ShapeDtypeStruct((n, nclass), jnp.float32),
        scratch_shapes=[
            pltpu.VMEM((n, nhid), jnp.float32),
            pltpu.VMEM((n, nclass), jnp.float32),
        ],
        compiler_params=pltpu.CompilerParams(
            dimension_semantics=("arbitrary", "arbitrary"),
        ),
    )(x, W1, b1.reshape(1, -1), W2, b2.reshape(1, -1), adj)


# speedup vs baseline: 1.0325x; 1.0067x over previous
import jax
import jax.numpy as jnp
from jax.experimental import pallas as pl
from jax.experimental.pallas import tpu as pltpu

_BM = 400


def _gcn_kernel(
    x_ref, w1_ref, b1_ref, w2_ref, b2_ref, adj_ref, out_ref, s1_scr, s2_scr
):
    p = pl.program_id(0)
    m = pl.program_id(1)

    @pl.when((p == 0) & (m == 0))
    def _():
        s1_scr[...] = jnp.dot(
            x_ref[...], w1_ref[...], preferred_element_type=jnp.float32
        )

    @pl.when(p == 0)
    def _():
        h = jnp.dot(
            adj_ref[...], s1_scr[...], preferred_element_type=jnp.float32
        )
        h = jnp.maximum(h + b1_ref[...], 0.0)
        s2_scr[pl.ds(m * _BM, _BM), :] = jnp.dot(
            h, w2_ref[...], preferred_element_type=jnp.float32
        )

    @pl.when(p == 1)
    def _():
        t = (
            jnp.dot(
                adj_ref[...], s2_scr[...], preferred_element_type=jnp.float32
            )
            + b2_ref[...]
        )
        mx = jnp.max(t, axis=1, keepdims=True)
        lse = jnp.log(jnp.sum(jnp.exp(t - mx), axis=1, keepdims=True)) + mx
        out_ref[...] = t - lse


def kernel(x, adj, W1, b1, W2, b2):
    n, nfeat = x.shape
    nhid = W1.shape[1]
    nclass = W2.shape[1]
    const = lambda p, m: (0, 0)
    rows = lambda p, m: (m, 0)
    out_rows = lambda p, m: (m * p, 0)

    return pl.pallas_call(
        _gcn_kernel,
        grid=(2, n // _BM),
        in_specs=[
            pl.BlockSpec((n, nfeat), const),
            pl.BlockSpec((nfeat, nhid), const),
            pl.BlockSpec((1, nhid), const),
            pl.BlockSpec((nhid, nclass), const),
            pl.BlockSpec((1, nclass), const),
            pl.BlockSpec((_BM, n), rows),
        ],
        out_specs=pl.BlockSpec((_BM, nclass), out_rows),
        out_shape=jax.ShapeDtypeStruct((n, nclass), jnp.float32),
        scratch_shapes=[
            pltpu.VMEM((n, nhid), jnp.float32),
            pltpu.VMEM((n, nclass), jnp.float32),
        ],
        compiler_params=pltpu.CompilerParams(
            dimension_semantics=("arbitrary", "arbitrary"),
            vmem_limit_bytes=64 * 1024 * 1024,
        ),
    )(x, W1, b1.reshape(1, -1), W2, b2.reshape(1, -1), adj)
